# baseline (speedup 1.0000x reference)
import functools

import jax
import jax.numpy as jnp
from jax import lax
from jax.experimental import pallas as pl
from jax.experimental.pallas import tpu as pltpu
from jax.experimental.pallas import tpu_sc as plsc

_B, _S, _H = 16, 2048, 1024
_L, _M = 32, 4
_LANES = 16


def _tc_body(spans_ref, seq_ref, means_ref):
    sp = spans_ref[0]
    s = sp[:, 0:1]
    e = sp[:, 1:2]

    pos = jax.lax.broadcasted_iota(jnp.int32, (_L * _M, _S), 1)
    maskf = ((pos >= s) & (pos <= e)).astype(jnp.bfloat16)

    x = seq_ref[0].astype(jnp.bfloat16)
    sums = jax.lax.dot_general(
        maskf, x,
        dimension_numbers=(((1,), (0,)), ((), ())),
        preferred_element_type=jnp.float32,
    )

    counts = (e - s + 1).astype(jnp.float32)
    means_ref[0] = sums / counts


def _round_bf16(v):
    t = v * jnp.float32(65537.0)
    return t - (t - v)


def _butterfly(v, op):
    lane = lax.broadcasted_iota(jnp.int32, (_LANES,), 0)
    for sh in (1, 2, 4, 8):
        idx = lane ^ sh
        v = op(v, v.at[idx].get(mode="promise_in_bounds"))
    return v


def _sc_body(means_hbm, w_hbm, b_hbm, logits_hbm, hid_hbm, best_hbm,
             rows_v, w_v, hid_v, log_v, b_v, best_v):
    wid = lax.axis_index("s") * 2 + lax.axis_index("c")

    @pl.when(wid < _B)
    def _():
        bb = wid
        pltpu.sync_copy(w_hbm.at[0], w_v)
        pltpu.sync_copy(b_hbm, b_v)
        lane = lax.broadcasted_iota(jnp.int32, (_LANES,), 0)
        zero = jnp.zeros((_LANES,), jnp.float32)
        bvec = b_v[...]

        for g in range(_L // _LANES):
            def link16(i, lacc, g=g):
                l = g * _LANES + i
                pltpu.sync_copy(means_hbm.at[bb, pl.ds(l * _M, _M)], rows_v)
                acc0 = jnp.where(lane == 0, bvec, zero)

                def chunk(c, acc):
                    off = c * _LANES
                    r0 = rows_v[0, pl.ds(off, _LANES)]
                    r1 = rows_v[1, pl.ds(off, _LANES)]
                    r2 = rows_v[2, pl.ds(off, _LANES)]
                    r3 = rows_v[3, pl.ds(off, _LANES)]
                    mx = jnp.maximum(jnp.maximum(r0, r1),
                                     jnp.maximum(r2, r3))
                    hid_v[pl.ds(off, _LANES)] = mx
                    wb = _round_bf16(w_v[pl.ds(off, _LANES)])
                    return acc + _round_bf16(mx) * wb

                acc = lax.fori_loop(0, _H // _LANES, chunk, acc0)
                pltpu.sync_copy(hid_v, hid_hbm.at[bb, l])
                logit = _butterfly(acc, jnp.add)
                return jnp.where(lane == i, logit, lacc)

            lvec = lax.fori_loop(0, _LANES, link16, zero)
            log_v[pl.ds(g * _LANES, _LANES)] = lvec

        pltpu.sync_copy(log_v, logits_hbm.at[bb])

        v0 = log_v[pl.ds(0, _LANES)]
        v1 = log_v[pl.ds(_LANES, _LANES)]
        wv = jnp.maximum(v0, v1)
        wi = jnp.where(v0 >= v1, lane, lane + _LANES)
        m = _butterfly(wv, jnp.maximum)
        cand = jnp.where(wv == m, wi, jnp.int32(2**30))
        bi = _butterfly(cand, jnp.minimum)
        best_v[...] = bi
        pltpu.sync_copy(best_v, best_hbm.at[bb])


def kernel(seq_hiddens, links_spans, W, b):
    B, S, H = seq_hiddens.shape
    L, M = links_spans.shape[1], links_spans.shape[2]
    sp = links_spans.reshape(B, L * M, 2).astype(jnp.int32)

    means = pl.pallas_call(
        _tc_body,
        grid=(B,),
        in_specs=[
            pl.BlockSpec((1, L * M, 2), lambda bb: (bb, 0, 0)),
            pl.BlockSpec((1, S, H), lambda bb: (bb, 0, 0)),
        ],
        out_specs=pl.BlockSpec((1, L * M, H), lambda bb: (bb, 0, 0)),
        out_shape=jax.ShapeDtypeStruct((B, L * M, H), jnp.float32),
        compiler_params=pltpu.CompilerParams(
            dimension_semantics=("arbitrary",),
        ),
    )(sp, seq_hiddens)

    b16 = jnp.broadcast_to(b.astype(jnp.float32), (_LANES,))

    sc = functools.partial(
        pl.kernel,
        mesh=plsc.VectorSubcoreMesh(core_axis_name="c", subcore_axis_name="s"),
        out_type=[
            jax.ShapeDtypeStruct((B, L), jnp.float32),
            jax.ShapeDtypeStruct((B, L, H), jnp.float32),
            jax.ShapeDtypeStruct((B, _LANES), jnp.int32),
        ],
        scratch_types=[
            pltpu.VMEM((M, H), jnp.float32),
            pltpu.VMEM((H,), jnp.float32),
            pltpu.VMEM((H,), jnp.float32),
            pltpu.VMEM((L,), jnp.float32),
            pltpu.VMEM((_LANES,), jnp.float32),
            pltpu.VMEM((_LANES,), jnp.int32),
        ],
    )(_sc_body)

    link_logits, link_hiddens, best_pad = sc(means, W, b16)
    return (link_logits, link_hiddens, best_pad[:, 0])


# speedup vs baseline: 2.3658x; 2.3658x over previous
import jax
import jax.numpy as jnp
from jax.experimental import pallas as pl
from jax.experimental.pallas import tpu as pltpu

_B, _S, _H = 16, 2048, 1024
_L, _M = 32, 4


def _body(spans_ref, seq_ref, w_ref, bias_ref,
          logits_ref, hid_ref, best_ref):
    bb = pl.program_id(0)

    sp = spans_ref[0]
    s = sp[:, 0:1]
    e = sp[:, 1:2]

    pos = jax.lax.broadcasted_iota(jnp.int32, (_L * _M, _S), 1)
    maskf = ((pos >= s) & (pos <= e)).astype(jnp.bfloat16)

    x = seq_ref[0].astype(jnp.bfloat16)
    sums = jax.lax.dot_general(
        maskf, x,
        dimension_numbers=(((1,), (0,)), ((), ())),
        preferred_element_type=jnp.float32,
    )

    counts = (e - s + 1).astype(jnp.float32)
    means = sums / counts
    hid = jnp.maximum(
        jnp.maximum(means[0 * _L:1 * _L], means[1 * _L:2 * _L]),
        jnp.maximum(means[2 * _L:3 * _L], means[3 * _L:4 * _L]),
    )
    hid_ref[0] = hid

    logits = jax.lax.dot_general(
        w_ref[...].astype(jnp.bfloat16), hid.astype(jnp.bfloat16),
        dimension_numbers=(((1,), (1,)), ((), ())),
        preferred_element_type=jnp.float32,
    ) + bias_ref[0]

    maxv = jnp.max(logits, axis=1, keepdims=True)
    ii = jax.lax.broadcasted_iota(jnp.int32, (1, _L), 1)
    best = jnp.min(jnp.where(logits == maxv, ii, jnp.int32(2**30)),
                   axis=1, keepdims=True)
    logits_ref[pl.ds(bb, 1), :] = logits
    best_ref[bb] = best[0, 0]


def kernel(seq_hiddens, links_spans, W, b):
    B, S, H = seq_hiddens.shape
    L, M = links_spans.shape[1], links_spans.shape[2]
    sp = jnp.transpose(links_spans, (0, 2, 1, 3)).reshape(B, M * L, 2)
    sp = sp.astype(jnp.int32)

    out = pl.pallas_call(
        _body,
        grid=(B,),
        in_specs=[
            pl.BlockSpec((1, M * L, 2), lambda bb: (bb, 0, 0)),
            pl.BlockSpec((1, S, H), lambda bb: (bb, 0, 0)),
            pl.BlockSpec((1, H), lambda bb: (0, 0)),
            pl.BlockSpec(memory_space=pltpu.SMEM),
        ],
        out_specs=[
            pl.BlockSpec((B, L), lambda bb: (0, 0)),
            pl.BlockSpec((1, L, H), lambda bb: (bb, 0, 0)),
            pl.BlockSpec(memory_space=pltpu.SMEM),
        ],
        out_shape=[
            jax.ShapeDtypeStruct((B, L), jnp.float32),
            jax.ShapeDtypeStruct((B, L, H), jnp.float32),
            jax.ShapeDtypeStruct((B,), jnp.int32),
        ],
        compiler_params=pltpu.CompilerParams(
            dimension_semantics=("parallel",),
        ),
    )(sp, seq_hiddens, W, b)

    return (out[0], out[1], out[2])
